# baseline (speedup 1.0000x reference)
import functools

import jax
import jax.numpy as jnp
from jax import lax
from jax.experimental import pallas as pl
from jax.experimental.pallas import tpu as pltpu
from jax.experimental.pallas import tpu_sc as plsc

N = 10000
E = 160000
D = 256
C = 64
F = 16
NP = 10240
EP = 163840
CH = 128
NROW = EP // CH
NCORE = 2
NSUB = 16
NT = NCORE * NSUB
EPT = EP // NT
NCH = EPT // CH
RPT = NP // NSUB
NBATCH = 8
BCH = NCH // NBATCH
PADROW = N + 100

_mesh = plsc.VectorSubcoreMesh(
    core_axis_name="c", subcore_axis_name="s",
    num_cores=NCORE, num_subcores=NSUB)


def _tile_ids():
    cid = lax.axis_index("c")
    sid = lax.axis_index("s")
    return cid, sid, cid * NSUB + sid


def _zero_acc_slice(zbuf, acc, sid):
    z = jnp.zeros((F,), jnp.float32)

    def zb(i, carry):
        zbuf[i] = z
        return carry

    lax.fori_loop(0, CH, zb, 0)
    base = sid * RPT
    for r in range(RPT // CH):
        pltpu.sync_copy(zbuf, acc.at[pl.ds(base + r * CH, CH)])


@functools.partial(
    pl.kernel,
    out_type=jax.ShapeDtypeStruct((NCORE, NP, F), jnp.float32),
    mesh=_mesh,
    compiler_params=pltpu.CompilerParams(use_tc_tiling_on_sc=False),
    scratch_types=[
        pltpu.VMEM((NCH, CH), jnp.int32),
        pltpu.VMEM((NCH, CH), jnp.int32),
        pltpu.VMEM((EPT, F), jnp.float32),
        pltpu.VMEM((CH, F), jnp.float32),
        pltpu.VMEM_SHARED((NP, F), jnp.float32),
        pltpu.VMEM_SHARED((NP, F), jnp.float32),
        pltpu.SemaphoreType.DMA,
        pltpu.SemaphoreType.DMA,
        pltpu.SemaphoreType.DMA,
    ],
)
def _sc_propagate(g_hbm, src_hbm, dst_hbm, out_hbm,
                  srcv, dstv, rows, zbuf, acc, gtab, sem_g, sem_g2, sem_s):
    cid, sid, tid = _tile_ids()
    pltpu.sync_copy(src_hbm.at[pl.ds(tid * NCH, NCH)], srcv)
    pltpu.sync_copy(dst_hbm.at[pl.ds(tid * NCH, NCH)], dstv)
    base = sid * RPT
    pltpu.async_copy(g_hbm.at[pl.ds(base, RPT)], gtab.at[pl.ds(base, RPT)],
                     sem_g)
    _zero_acc_slice(zbuf, acc, sid)
    pltpu.make_async_copy(g_hbm.at[pl.ds(base, RPT)],
                          gtab.at[pl.ds(base, RPT)], sem_g).wait()
    plsc.subcore_barrier()

    def fire_gathers(k, sem):
        def fg(j, carry):
            pltpu.async_copy(gtab.at[srcv.at[j]],
                             rows.at[pl.ds(j * CH, CH)], sem)
            return carry
        lax.fori_loop(k * BCH, (k + 1) * BCH, fg, 0)

    def wait_gathers(sem):
        pltpu.make_async_copy(g_hbm.at[pl.ds(0, BCH * CH)],
                              rows.at[pl.ds(0, BCH * CH)], sem).wait()

    sems = (sem_g, sem_g2)
    fire_gathers(0, sems[0])
    fire_gathers(1, sems[1])
    for k in range(NBATCH):
        wait_gathers(sems[k % 2])
        if k + 2 < NBATCH:
            fire_gathers(k + 2, sems[k % 2])

        def fs(j, carry):
            pltpu.async_copy(rows.at[pl.ds(j * CH, CH)], acc.at[dstv.at[j]],
                             sem_s, add=True)
            return carry
        lax.fori_loop(k * BCH, (k + 1) * BCH, fs, 0)
    pltpu.make_async_copy(rows, acc.at[pl.ds(0, EPT)], sem_s).wait()
    plsc.subcore_barrier()
    base = sid * RPT
    pltpu.sync_copy(acc.at[pl.ds(base, RPT)],
                    out_hbm.at[cid, pl.ds(base, RPT)])


@functools.partial(
    pl.kernel,
    out_type=jax.ShapeDtypeStruct((NCORE, NP, F), jnp.float32),
    mesh=_mesh,
    compiler_params=pltpu.CompilerParams(use_tc_tiling_on_sc=False),
    scratch_types=[
        pltpu.VMEM((NCH, CH), jnp.int32),
        pltpu.VMEM((CH, F), jnp.float32),
        pltpu.VMEM((CH, F), jnp.float32),
        pltpu.VMEM_SHARED((NP, F), jnp.float32),
        pltpu.SemaphoreType.DMA,
    ],
)
def _sc_degree(dst_hbm, out_hbm, dstv, ones_b, zbuf, acc, sem_s):
    cid, sid, tid = _tile_ids()
    pltpu.sync_copy(dst_hbm.at[pl.ds(tid * NCH, NCH)], dstv)
    one = jnp.ones((F,), jnp.float32)

    def ob(i, carry):
        ones_b[i] = one
        return carry

    lax.fori_loop(0, CH, ob, 0)
    _zero_acc_slice(zbuf, acc, sid)
    plsc.subcore_barrier()

    def fire_s(j, carry):
        pltpu.async_copy(ones_b, acc.at[dstv.at[j]], sem_s, add=True)
        return carry

    lax.fori_loop(0, NCH, fire_s, 0)

    def drain_s(j, carry):
        pltpu.make_async_copy(ones_b, acc.at[dstv.at[j]], sem_s).wait()
        return carry

    lax.fori_loop(0, NCH, drain_s, 0)
    plsc.subcore_barrier()
    base = sid * RPT
    pltpu.sync_copy(acc.at[pl.ds(base, RPT)],
                    out_hbm.at[cid, pl.ds(base, RPT)])


@functools.partial(
    pl.kernel,
    out_type=jax.ShapeDtypeStruct((NCORE, NP, F), jnp.float32),
    mesh=_mesh,
    compiler_params=pltpu.CompilerParams(use_tc_tiling_on_sc=False),
    scratch_types=[
        pltpu.VMEM((NCH, CH), jnp.int32),
        pltpu.VMEM((NCH, CH), jnp.int32),
        pltpu.VMEM((EPT, F), jnp.float32),
        pltpu.VMEM((CH, F), jnp.float32),
        pltpu.VMEM_SHARED((NP, F), jnp.float32),
        pltpu.VMEM_SHARED((NP, F), jnp.float32),
        pltpu.SemaphoreType.DMA,
        pltpu.SemaphoreType.DMA,
        pltpu.SemaphoreType.DMA,
    ],
)
def _sc_propagate_fused(s_hbm, d2_hbm, p_hbm, src_hbm, dst_hbm, out_hbm,
                        srcv, dstv, rows, zbuf, acc, gtab,
                        sem_g, sem_g2, sem_s):
    cid, sid, tid = _tile_ids()
    pltpu.sync_copy(src_hbm.at[pl.ds(tid * NCH, NCH)], srcv)
    pltpu.sync_copy(dst_hbm.at[pl.ds(tid * NCH, NCH)], dstv)
    base = sid * RPT
    pltpu.async_copy(s_hbm.at[pl.ds(base, RPT)], rows.at[pl.ds(0, RPT)], sem_g)
    pltpu.async_copy(d2_hbm.at[pl.ds(base, RPT)], rows.at[pl.ds(RPT, RPT)],
                     sem_g)
    pltpu.async_copy(p_hbm.at[0, pl.ds(base, RPT)],
                     rows.at[pl.ds(2 * RPT, RPT)], sem_g)
    pltpu.async_copy(p_hbm.at[1, pl.ds(base, RPT)],
                     rows.at[pl.ds(3 * RPT, RPT)], sem_g)
    _zero_acc_slice(zbuf, acc, sid)
    pltpu.make_async_copy(s_hbm.at[pl.ds(0, 4 * RPT)],
                          rows.at[pl.ds(0, 4 * RPT)], sem_g).wait()

    def comb(i, carry):
        for u in range(4):
            r = 4 * i + u
            rows[4 * RPT + r] = rows[r] + rows[RPT + r] * (
                rows[2 * RPT + r] + rows[3 * RPT + r])
        return carry

    lax.fori_loop(0, RPT // 4, comb, 0)
    pltpu.sync_copy(rows.at[pl.ds(4 * RPT, RPT)], gtab.at[pl.ds(base, RPT)])
    plsc.subcore_barrier()

    def fire_gathers(k, sem):
        def fg(j, carry):
            pltpu.async_copy(gtab.at[srcv.at[j]],
                             rows.at[pl.ds(j * CH, CH)], sem)
            return carry
        lax.fori_loop(k * BCH, (k + 1) * BCH, fg, 0)

    def wait_gathers(sem):
        pltpu.make_async_copy(s_hbm.at[pl.ds(0, BCH * CH)],
                              rows.at[pl.ds(0, BCH * CH)], sem).wait()

    sems = (sem_g, sem_g2)
    fire_gathers(0, sems[0])
    fire_gathers(1, sems[1])
    for k in range(NBATCH):
        wait_gathers(sems[k % 2])
        if k + 2 < NBATCH:
            fire_gathers(k + 2, sems[k % 2])

        def fs(j, carry):
            pltpu.async_copy(rows.at[pl.ds(j * CH, CH)], acc.at[dstv.at[j]],
                             sem_s, add=True)
            return carry
        lax.fori_loop(k * BCH, (k + 1) * BCH, fs, 0)
    pltpu.make_async_copy(rows, acc.at[pl.ds(0, EPT)], sem_s).wait()
    plsc.subcore_barrier()
    pltpu.sync_copy(acc.at[pl.ds(base, RPT)],
                    out_hbm.at[cid, pl.ds(base, RPT)])


def _leaky(v):
    return jnp.where(v >= 0, v, 0.02 * v)


BM = 1024


def _matmul_body(x_ref, w_ref, y_ref):
    y_ref[...] = jnp.dot(x_ref[...], w_ref[...],
                         preferred_element_type=jnp.float32)


_matmul1 = pl.pallas_call(
    _matmul_body,
    grid=(NP // BM,),
    in_specs=[
        pl.BlockSpec((BM, D), lambda i: (i, 0)),
        pl.BlockSpec((D, 4 * F), lambda i: (0, 0)),
    ],
    out_specs=pl.BlockSpec((BM, 4 * F), lambda i: (i, 0)),
    out_shape=jax.ShapeDtypeStruct((NP, 4 * F), jnp.float32),
)


def _prep_body(degp_ref, y_ref, y0_ref, s1_ref, s2_ref, g3_ref,
               d1_ref, d2_ref):
    deg = degp_ref[0] + degp_ref[1]
    dinv = jnp.where(deg > 0.0, lax.rsqrt(jnp.maximum(deg, 1e-12)), 0.0)
    y = y_ref[...]
    y0_ref[...] = y[:, 0:16]
    s1_ref[...] = y[:, 16:32] * dinv
    s2_ref[...] = y[:, 32:48] * dinv
    g3_ref[...] = y[:, 48:64] * dinv
    d1_ref[...] = dinv
    d2_ref[...] = dinv * dinv


_prep = pl.pallas_call(
    _prep_body,
    grid=(NP // (2 * BM),),
    in_specs=[
        pl.BlockSpec((NCORE, 2 * BM, F), lambda i: (0, i, 0)),
        pl.BlockSpec((2 * BM, 4 * F), lambda i: (i, 0)),
    ],
    out_specs=[pl.BlockSpec((2 * BM, F), lambda i: (i, 0))] * 6,
    out_shape=[jax.ShapeDtypeStruct((NP, F), jnp.float32)] * 6,
)


def _finish_proj_body(y0_ref, d1_ref, p_ref, b_ref, w_ref,
                      o0_ref, o1_ref, o2_ref, o3_ref):
    h = y0_ref[...] + d1_ref[...] * (p_ref[0] + p_ref[1]) + b_ref[...]
    h = _leaky(h)
    y = jnp.dot(h, w_ref[...], preferred_element_type=jnp.float32)
    dinv = d1_ref[...]
    o0_ref[...] = y[:, 0:16]
    o1_ref[...] = y[:, 16:32] * dinv
    o2_ref[...] = y[:, 32:48] * dinv
    o3_ref[...] = y[:, 48:64] * dinv


_finish_proj = pl.pallas_call(
    _finish_proj_body,
    grid=(NP // (2 * BM),),
    in_specs=[
        pl.BlockSpec((2 * BM, F), lambda i: (i, 0)),
        pl.BlockSpec((2 * BM, F), lambda i: (i, 0)),
        pl.BlockSpec((NCORE, 2 * BM, F), lambda i: (0, i, 0)),
        pl.BlockSpec((1, F), lambda i: (0, 0)),
        pl.BlockSpec((F, 4 * F), lambda i: (0, 0)),
    ],
    out_specs=[pl.BlockSpec((2 * BM, F), lambda i: (i, 0))] * 4,
    out_shape=[jax.ShapeDtypeStruct((NP, F), jnp.float32)] * 4,
)


def _final_body(y0_ref, d1_ref, p_ref, b3_ref,
                w0_ref, b0_ref, w1_ref, b1_ref, w2_ref, b2_ref, o_ref):
    h = y0_ref[...] + d1_ref[...] * (p_ref[0] + p_ref[1]) + b3_ref[...]
    t = _leaky(jnp.dot(h, w0_ref[...], preferred_element_type=jnp.float32)
               + b0_ref[...])
    t = _leaky(jnp.dot(t, w1_ref[...], preferred_element_type=jnp.float32)
               + b1_ref[...])
    o = (jnp.dot(t, w2_ref[...], preferred_element_type=jnp.float32)
         + b2_ref[...])
    m = jnp.max(o, axis=1, keepdims=True)
    e = o - m
    lse = jnp.log(jnp.sum(jnp.exp(e), axis=1, keepdims=True))
    o_ref[...] = e - lse


_final = pl.pallas_call(
    _final_body,
    grid=(NP // (2 * BM),),
    in_specs=[
        pl.BlockSpec((2 * BM, F), lambda i: (i, 0)),
        pl.BlockSpec((2 * BM, F), lambda i: (i, 0)),
        pl.BlockSpec((NCORE, 2 * BM, F), lambda i: (0, i, 0)),
        pl.BlockSpec((1, F), lambda i: (0, 0)),
        pl.BlockSpec((F, 4 * F), lambda i: (0, 0)),
        pl.BlockSpec((1, 4 * F), lambda i: (0, 0)),
        pl.BlockSpec((4 * F, F), lambda i: (0, 0)),
        pl.BlockSpec((1, F), lambda i: (0, 0)),
        pl.BlockSpec((F, C), lambda i: (0, 0)),
        pl.BlockSpec((1, C), lambda i: (0, 0)),
    ],
    out_specs=pl.BlockSpec((2 * BM, C), lambda i: (i, 0)),
    out_shape=jax.ShapeDtypeStruct((NP, C), jnp.float32),
)


def kernel(x, edge_index, conv1_W, conv1_b, conv2_W, conv2_b,
           conv3_W, conv3_b, mlp_W0, mlp_b0, mlp_W1, mlp_b1, mlp_W2, mlp_b2):
    src = edge_index[0].astype(jnp.int32)
    dst = edge_index[1].astype(jnp.int32)
    epad = jnp.full((EP - E,), PADROW, jnp.int32)
    src2 = jnp.concatenate([src, epad]).reshape(NROW, CH)
    dst2 = jnp.concatenate([dst, epad]).reshape(NROW, CH)
    x_pad = jnp.pad(x, ((0, NP - N), (0, 0)))
    w1 = jnp.concatenate([conv1_W[k] for k in range(4)], axis=1)
    w2 = jnp.concatenate([conv2_W[k] for k in range(4)], axis=1)
    w3 = jnp.concatenate([conv3_W[k] for k in range(4)], axis=1)

    y1 = _matmul1(x_pad, w1)
    degp = _sc_degree(dst2)
    y0, s1, s2, g3, d1, d2 = _prep(degp, y1)
    for wc, bc in ((w2, conv1_b), (w3, conv2_b)):
        p = _sc_propagate(g3, src2, dst2)
        p = _sc_propagate_fused(s2, d2, p, src2, dst2)
        p = _sc_propagate_fused(s1, d2, p, src2, dst2)
        y0, s1, s2, g3 = _finish_proj(y0, d1, p, bc.reshape(1, F), wc)
    p = _sc_propagate(g3, src2, dst2)
    p = _sc_propagate_fused(s2, d2, p, src2, dst2)
    p = _sc_propagate_fused(s1, d2, p, src2, dst2)
    out = _final(y0, d1, p, conv3_b.reshape(1, F),
                 mlp_W0, mlp_b0.reshape(1, 4 * F),
                 mlp_W1, mlp_b1.reshape(1, F),
                 mlp_W2, mlp_b2.reshape(1, C))
    return out[:N]


# speedup vs baseline: 1.0139x; 1.0139x over previous
import functools

import jax
import jax.numpy as jnp
from jax import lax
from jax.experimental import pallas as pl
from jax.experimental.pallas import tpu as pltpu
from jax.experimental.pallas import tpu_sc as plsc

N = 10000
E = 160000
D = 256
C = 64
F = 16
NP = 10240
EP = 163840
CH = 128
NROW = EP // CH
NCORE = 2
NSUB = 16
NT = NCORE * NSUB
EPT = EP // NT
NCH = EPT // CH
RPT = NP // NSUB
NBATCH = 8
BCH = NCH // NBATCH
PADROW = N + 100

_mesh = plsc.VectorSubcoreMesh(
    core_axis_name="c", subcore_axis_name="s",
    num_cores=NCORE, num_subcores=NSUB)


def _tile_ids():
    cid = lax.axis_index("c")
    sid = lax.axis_index("s")
    return cid, sid, cid * NSUB + sid


def _zero_acc_slice(zbuf, acc, sid):
    z = jnp.zeros((F,), jnp.float32)

    def zb(i, carry):
        zbuf[i] = z
        return carry

    lax.fori_loop(0, CH, zb, 0)
    base = sid * RPT
    for r in range(RPT // CH):
        pltpu.sync_copy(zbuf, acc.at[pl.ds(base + r * CH, CH)])


@functools.partial(
    pl.kernel,
    out_type=jax.ShapeDtypeStruct((NCORE, NP, F), jnp.float32),
    mesh=_mesh,
    compiler_params=pltpu.CompilerParams(use_tc_tiling_on_sc=False),
    scratch_types=[
        pltpu.VMEM((NCH, CH), jnp.int32),
        pltpu.VMEM((NCH, CH), jnp.int32),
        pltpu.VMEM((EPT, F), jnp.float32),
        pltpu.VMEM((CH, F), jnp.float32),
        pltpu.VMEM_SHARED((NP, F), jnp.float32),
        pltpu.VMEM_SHARED((NP, F), jnp.float32),
        pltpu.SemaphoreType.DMA,
        pltpu.SemaphoreType.DMA,
        pltpu.SemaphoreType.DMA,
    ],
)
def _sc_propagate(g_hbm, src_hbm, dst_hbm, out_hbm,
                  srcv, dstv, rows, zbuf, acc, gtab, sem_g, sem_g2, sem_s):
    cid, sid, tid = _tile_ids()
    pltpu.sync_copy(src_hbm.at[pl.ds(tid * NCH, NCH)], srcv)
    pltpu.sync_copy(dst_hbm.at[pl.ds(tid * NCH, NCH)], dstv)
    base = sid * RPT
    pltpu.async_copy(g_hbm.at[pl.ds(base, RPT)], gtab.at[pl.ds(base, RPT)],
                     sem_g)
    _zero_acc_slice(zbuf, acc, sid)
    pltpu.make_async_copy(g_hbm.at[pl.ds(base, RPT)],
                          gtab.at[pl.ds(base, RPT)], sem_g).wait()
    plsc.subcore_barrier()

    def fire_gathers(k, sem):
        def fg(j, carry):
            pltpu.async_copy(gtab.at[srcv.at[j]],
                             rows.at[pl.ds(j * CH, CH)], sem)
            return carry
        lax.fori_loop(k * BCH, (k + 1) * BCH, fg, 0)

    def wait_gathers(sem):
        pltpu.make_async_copy(g_hbm.at[pl.ds(0, BCH * CH)],
                              rows.at[pl.ds(0, BCH * CH)], sem).wait()

    sems = (sem_g, sem_g2)
    fire_gathers(0, sems[0])
    fire_gathers(1, sems[1])
    for k in range(NBATCH):
        wait_gathers(sems[k % 2])
        if k + 2 < NBATCH:
            fire_gathers(k + 2, sems[k % 2])

        def fs(j, carry):
            pltpu.async_copy(rows.at[pl.ds(j * CH, CH)], acc.at[dstv.at[j]],
                             sem_s, add=True)
            return carry
        lax.fori_loop(k * BCH, (k + 1) * BCH, fs, 0)
    pltpu.make_async_copy(rows, acc.at[pl.ds(0, EPT)], sem_s).wait()
    plsc.subcore_barrier()
    base = sid * RPT
    pltpu.sync_copy(acc.at[pl.ds(base, RPT)],
                    out_hbm.at[cid, pl.ds(base, RPT)])


@functools.partial(
    pl.kernel,
    out_type=jax.ShapeDtypeStruct((NCORE, NP, F), jnp.float32),
    mesh=_mesh,
    compiler_params=pltpu.CompilerParams(use_tc_tiling_on_sc=False),
    scratch_types=[
        pltpu.VMEM((NCH, CH), jnp.int32),
        pltpu.VMEM((CH, F), jnp.float32),
        pltpu.VMEM((CH, F), jnp.float32),
        pltpu.VMEM_SHARED((NP, F), jnp.float32),
        pltpu.SemaphoreType.DMA,
    ],
)
def _sc_degree(dst_hbm, out_hbm, dstv, ones_b, zbuf, acc, sem_s):
    cid, sid, tid = _tile_ids()
    pltpu.sync_copy(dst_hbm.at[pl.ds(tid * NCH, NCH)], dstv)
    one = jnp.ones((F,), jnp.float32)

    def ob(i, carry):
        ones_b[i] = one
        return carry

    lax.fori_loop(0, CH, ob, 0)
    _zero_acc_slice(zbuf, acc, sid)
    plsc.subcore_barrier()

    def fire_s(j, carry):
        pltpu.async_copy(ones_b, acc.at[dstv.at[j]], sem_s, add=True)
        return carry

    lax.fori_loop(0, NCH, fire_s, 0)

    def drain_s(j, carry):
        pltpu.make_async_copy(ones_b, acc.at[dstv.at[j]], sem_s).wait()
        return carry

    lax.fori_loop(0, NCH, drain_s, 0)
    plsc.subcore_barrier()
    base = sid * RPT
    pltpu.sync_copy(acc.at[pl.ds(base, RPT)],
                    out_hbm.at[cid, pl.ds(base, RPT)])


@functools.partial(
    pl.kernel,
    out_type=jax.ShapeDtypeStruct((NCORE, NP, F), jnp.float32),
    mesh=_mesh,
    compiler_params=pltpu.CompilerParams(use_tc_tiling_on_sc=False),
    scratch_types=[
        pltpu.VMEM((NCH, CH), jnp.int32),
        pltpu.VMEM((NCH, CH), jnp.int32),
        pltpu.VMEM((EPT, F), jnp.float32),
        pltpu.VMEM((CH, F), jnp.float32),
        pltpu.VMEM_SHARED((NP, F), jnp.float32),
        pltpu.VMEM_SHARED((NP, F), jnp.float32),
        pltpu.SemaphoreType.DMA,
        pltpu.SemaphoreType.DMA,
        pltpu.SemaphoreType.DMA,
    ],
)
def _sc_propagate_fused(s_hbm, d2_hbm, p_hbm, src_hbm, dst_hbm, out_hbm,
                        srcv, dstv, rows, zbuf, acc, gtab,
                        sem_g, sem_g2, sem_s):
    cid, sid, tid = _tile_ids()
    pltpu.sync_copy(src_hbm.at[pl.ds(tid * NCH, NCH)], srcv)
    pltpu.sync_copy(dst_hbm.at[pl.ds(tid * NCH, NCH)], dstv)
    base = sid * RPT
    pltpu.async_copy(s_hbm.at[pl.ds(base, RPT)], rows.at[pl.ds(0, RPT)], sem_g)
    pltpu.async_copy(d2_hbm.at[pl.ds(base, RPT)], rows.at[pl.ds(RPT, RPT)],
                     sem_g)
    pltpu.async_copy(p_hbm.at[0, pl.ds(base, RPT)],
                     rows.at[pl.ds(2 * RPT, RPT)], sem_g)
    pltpu.async_copy(p_hbm.at[1, pl.ds(base, RPT)],
                     rows.at[pl.ds(3 * RPT, RPT)], sem_g)
    _zero_acc_slice(zbuf, acc, sid)
    pltpu.make_async_copy(s_hbm.at[pl.ds(0, 4 * RPT)],
                          rows.at[pl.ds(0, 4 * RPT)], sem_g).wait()

    def comb(i, carry):
        for u in range(4):
            r = 4 * i + u
            rows[4 * RPT + r] = rows[r] + rows[RPT + r] * (
                rows[2 * RPT + r] + rows[3 * RPT + r])
        return carry

    lax.fori_loop(0, RPT // 4, comb, 0)
    pltpu.sync_copy(rows.at[pl.ds(4 * RPT, RPT)], gtab.at[pl.ds(base, RPT)])
    plsc.subcore_barrier()

    def fire_gathers(k, sem):
        def fg(j, carry):
            pltpu.async_copy(gtab.at[srcv.at[j]],
                             rows.at[pl.ds(j * CH, CH)], sem)
            return carry
        lax.fori_loop(k * BCH, (k + 1) * BCH, fg, 0)

    def wait_gathers(sem):
        pltpu.make_async_copy(s_hbm.at[pl.ds(0, BCH * CH)],
                              rows.at[pl.ds(0, BCH * CH)], sem).wait()

    sems = (sem_g, sem_g2)
    fire_gathers(0, sems[0])
    fire_gathers(1, sems[1])
    for k in range(NBATCH):
        wait_gathers(sems[k % 2])
        if k + 2 < NBATCH:
            fire_gathers(k + 2, sems[k % 2])

        def fs(j, carry):
            pltpu.async_copy(rows.at[pl.ds(j * CH, CH)], acc.at[dstv.at[j]],
                             sem_s, add=True)
            return carry
        lax.fori_loop(k * BCH, (k + 1) * BCH, fs, 0)
    pltpu.make_async_copy(rows, acc.at[pl.ds(0, EPT)], sem_s).wait()
    plsc.subcore_barrier()
    pltpu.sync_copy(acc.at[pl.ds(base, RPT)],
                    out_hbm.at[cid, pl.ds(base, RPT)])


def _leaky(v):
    return jnp.where(v >= 0, v, 0.02 * v)


BM = 1024


def _prep_body(degp_ref, x_ref, w_ref,
               y0_ref, s1_ref, s2_ref, g3_ref, d1_ref, d2_ref):
    deg = degp_ref[0] + degp_ref[1]
    dinv = jnp.where(deg > 0.0, lax.rsqrt(jnp.maximum(deg, 1e-12)), 0.0)
    y = jnp.dot(x_ref[...], w_ref[...], preferred_element_type=jnp.float32)
    y0_ref[...] = y[:, 0:16]
    s1_ref[...] = y[:, 16:32] * dinv
    s2_ref[...] = y[:, 32:48] * dinv
    g3_ref[...] = y[:, 48:64] * dinv
    d1_ref[...] = dinv
    d2_ref[...] = dinv * dinv


_prep = pl.pallas_call(
    _prep_body,
    grid=(NP // BM,),
    in_specs=[
        pl.BlockSpec((NCORE, BM, F), lambda i: (0, i, 0)),
        pl.BlockSpec((BM, D), lambda i: (i, 0)),
        pl.BlockSpec((D, 4 * F), lambda i: (0, 0)),
    ],
    out_specs=[pl.BlockSpec((BM, F), lambda i: (i, 0))] * 6,
    out_shape=[jax.ShapeDtypeStruct((NP, F), jnp.float32)] * 6,
)


def _finish_proj_body(y0_ref, d1_ref, p_ref, b_ref, w_ref,
                      o0_ref, o1_ref, o2_ref, o3_ref):
    h = y0_ref[...] + d1_ref[...] * (p_ref[0] + p_ref[1]) + b_ref[...]
    h = _leaky(h)
    y = jnp.dot(h, w_ref[...], preferred_element_type=jnp.float32)
    dinv = d1_ref[...]
    o0_ref[...] = y[:, 0:16]
    o1_ref[...] = y[:, 16:32] * dinv
    o2_ref[...] = y[:, 32:48] * dinv
    o3_ref[...] = y[:, 48:64] * dinv


_finish_proj = pl.pallas_call(
    _finish_proj_body,
    grid=(NP // (2 * BM),),
    in_specs=[
        pl.BlockSpec((2 * BM, F), lambda i: (i, 0)),
        pl.BlockSpec((2 * BM, F), lambda i: (i, 0)),
        pl.BlockSpec((NCORE, 2 * BM, F), lambda i: (0, i, 0)),
        pl.BlockSpec((1, F), lambda i: (0, 0)),
        pl.BlockSpec((F, 4 * F), lambda i: (0, 0)),
    ],
    out_specs=[pl.BlockSpec((2 * BM, F), lambda i: (i, 0))] * 4,
    out_shape=[jax.ShapeDtypeStruct((NP, F), jnp.float32)] * 4,
)


def _final_body(y0_ref, d1_ref, p_ref, b3_ref,
                w0_ref, b0_ref, w1_ref, b1_ref, w2_ref, b2_ref, o_ref):
    h = y0_ref[...] + d1_ref[...] * (p_ref[0] + p_ref[1]) + b3_ref[...]
    t = _leaky(jnp.dot(h, w0_ref[...], preferred_element_type=jnp.float32)
               + b0_ref[...])
    t = _leaky(jnp.dot(t, w1_ref[...], preferred_element_type=jnp.float32)
               + b1_ref[...])
    o = (jnp.dot(t, w2_ref[...], preferred_element_type=jnp.float32)
         + b2_ref[...])
    m = jnp.max(o, axis=1, keepdims=True)
    e = o - m
    lse = jnp.log(jnp.sum(jnp.exp(e), axis=1, keepdims=True))
    o_ref[...] = e - lse


_final = pl.pallas_call(
    _final_body,
    grid=(NP // (2 * BM),),
    in_specs=[
        pl.BlockSpec((2 * BM, F), lambda i: (i, 0)),
        pl.BlockSpec((2 * BM, F), lambda i: (i, 0)),
        pl.BlockSpec((NCORE, 2 * BM, F), lambda i: (0, i, 0)),
        pl.BlockSpec((1, F), lambda i: (0, 0)),
        pl.BlockSpec((F, 4 * F), lambda i: (0, 0)),
        pl.BlockSpec((1, 4 * F), lambda i: (0, 0)),
        pl.BlockSpec((4 * F, F), lambda i: (0, 0)),
        pl.BlockSpec((1, F), lambda i: (0, 0)),
        pl.BlockSpec((F, C), lambda i: (0, 0)),
        pl.BlockSpec((1, C), lambda i: (0, 0)),
    ],
    out_specs=pl.BlockSpec((2 * BM, C), lambda i: (i, 0)),
    out_shape=jax.ShapeDtypeStruct((NP, C), jnp.float32),
)


def kernel(x, edge_index, conv1_W, conv1_b, conv2_W, conv2_b,
           conv3_W, conv3_b, mlp_W0, mlp_b0, mlp_W1, mlp_b1, mlp_W2, mlp_b2):
    src = edge_index[0].astype(jnp.int32)
    dst = edge_index[1].astype(jnp.int32)
    epad = jnp.full((EP - E,), PADROW, jnp.int32)
    src2 = jnp.concatenate([src, epad]).reshape(NROW, CH)
    dst2 = jnp.concatenate([dst, epad]).reshape(NROW, CH)
    x_pad = jnp.pad(x, ((0, NP - N), (0, 0)))
    w1 = jnp.concatenate([conv1_W[k] for k in range(4)], axis=1)
    w2 = jnp.concatenate([conv2_W[k] for k in range(4)], axis=1)
    w3 = jnp.concatenate([conv3_W[k] for k in range(4)], axis=1)

    degp = _sc_degree(dst2)
    y0, s1, s2, g3, d1, d2 = _prep(degp, x_pad, w1)
    for wc, bc in ((w2, conv1_b), (w3, conv2_b)):
        p = _sc_propagate(g3, src2, dst2)
        p = _sc_propagate_fused(s2, d2, p, src2, dst2)
        p = _sc_propagate_fused(s1, d2, p, src2, dst2)
        y0, s1, s2, g3 = _finish_proj(y0, d1, p, bc.reshape(1, F), wc)
    p = _sc_propagate(g3, src2, dst2)
    p = _sc_propagate_fused(s2, d2, p, src2, dst2)
    p = _sc_propagate_fused(s1, d2, p, src2, dst2)
    out = _final(y0, d1, p, conv3_b.reshape(1, F),
                 mlp_W0, mlp_b0.reshape(1, 4 * F),
                 mlp_W1, mlp_b1.reshape(1, F),
                 mlp_W2, mlp_b2.reshape(1, C))
    return out[:N]


# speedup vs baseline: 1.0315x; 1.0174x over previous
import functools

import jax
import jax.numpy as jnp
from jax import lax
from jax.experimental import pallas as pl
from jax.experimental.pallas import tpu as pltpu
from jax.experimental.pallas import tpu_sc as plsc

N = 10000
E = 160000
D = 256
C = 64
F = 16
NP = 10240
EP = 163840
CH = 128
NROW = EP // CH
NCORE = 2
NSUB = 16
NT = NCORE * NSUB
EPT = EP // NT
NCH = EPT // CH
RPT = NP // NSUB
NBATCH = 20
BCH = NCH // NBATCH
PADROW = N + 100

_mesh = plsc.VectorSubcoreMesh(
    core_axis_name="c", subcore_axis_name="s",
    num_cores=NCORE, num_subcores=NSUB)


def _tile_ids():
    cid = lax.axis_index("c")
    sid = lax.axis_index("s")
    return cid, sid, cid * NSUB + sid


def _zero_acc_slice(zbuf, acc, sid):
    z = jnp.zeros((F,), jnp.float32)

    def zb(i, carry):
        zbuf[i] = z
        return carry

    lax.fori_loop(0, CH, zb, 0)
    base = sid * RPT
    for r in range(RPT // CH):
        pltpu.sync_copy(zbuf, acc.at[pl.ds(base + r * CH, CH)])


@functools.partial(
    pl.kernel,
    out_type=jax.ShapeDtypeStruct((NCORE, NP, F), jnp.float32),
    mesh=_mesh,
    compiler_params=pltpu.CompilerParams(use_tc_tiling_on_sc=False),
    scratch_types=[
        pltpu.VMEM((NCH, CH), jnp.int32),
        pltpu.VMEM((NCH, CH), jnp.int32),
        pltpu.VMEM((EPT, F), jnp.float32),
        pltpu.VMEM((CH, F), jnp.float32),
        pltpu.VMEM_SHARED((NP, F), jnp.float32),
        pltpu.VMEM_SHARED((NP, F), jnp.float32),
        pltpu.SemaphoreType.DMA,
        pltpu.SemaphoreType.DMA,
        pltpu.SemaphoreType.DMA,
    ],
)
def _sc_propagate(g_hbm, src_hbm, dst_hbm, out_hbm,
                  srcv, dstv, rows, zbuf, acc, gtab, sem_g, sem_g2, sem_s):
    cid, sid, tid = _tile_ids()
    pltpu.sync_copy(src_hbm.at[pl.ds(tid * NCH, NCH)], srcv)
    pltpu.sync_copy(dst_hbm.at[pl.ds(tid * NCH, NCH)], dstv)
    base = sid * RPT
    pltpu.async_copy(g_hbm.at[pl.ds(base, RPT)], gtab.at[pl.ds(base, RPT)],
                     sem_g)
    _zero_acc_slice(zbuf, acc, sid)
    pltpu.make_async_copy(g_hbm.at[pl.ds(base, RPT)],
                          gtab.at[pl.ds(base, RPT)], sem_g).wait()
    plsc.subcore_barrier()

    def fire_gathers(k, sem):
        def fg(j, carry):
            pltpu.async_copy(gtab.at[srcv.at[j]],
                             rows.at[pl.ds(j * CH, CH)], sem)
            return carry
        lax.fori_loop(k * BCH, (k + 1) * BCH, fg, 0)

    def wait_gathers(sem):
        pltpu.make_async_copy(g_hbm.at[pl.ds(0, BCH * CH)],
                              rows.at[pl.ds(0, BCH * CH)], sem).wait()

    sems = (sem_g, sem_g2)
    fire_gathers(0, sems[0])
    fire_gathers(1, sems[1])
    for k in range(NBATCH):
        wait_gathers(sems[k % 2])
        if k + 2 < NBATCH:
            fire_gathers(k + 2, sems[k % 2])

        def fs(j, carry):
            pltpu.async_copy(rows.at[pl.ds(j * CH, CH)], acc.at[dstv.at[j]],
                             sem_s, add=True)
            return carry
        lax.fori_loop(k * BCH, (k + 1) * BCH, fs, 0)
    pltpu.make_async_copy(rows, acc.at[pl.ds(0, EPT)], sem_s).wait()
    plsc.subcore_barrier()
    base = sid * RPT
    pltpu.sync_copy(acc.at[pl.ds(base, RPT)],
                    out_hbm.at[cid, pl.ds(base, RPT)])


@functools.partial(
    pl.kernel,
    out_type=jax.ShapeDtypeStruct((NCORE, NP, F), jnp.float32),
    mesh=_mesh,
    compiler_params=pltpu.CompilerParams(use_tc_tiling_on_sc=False),
    scratch_types=[
        pltpu.VMEM((NCH, CH), jnp.int32),
        pltpu.VMEM((CH, F), jnp.float32),
        pltpu.VMEM((CH, F), jnp.float32),
        pltpu.VMEM_SHARED((NP, F), jnp.float32),
        pltpu.SemaphoreType.DMA,
    ],
)
def _sc_degree(dst_hbm, out_hbm, dstv, ones_b, zbuf, acc, sem_s):
    cid, sid, tid = _tile_ids()
    pltpu.sync_copy(dst_hbm.at[pl.ds(tid * NCH, NCH)], dstv)
    one = jnp.ones((F,), jnp.float32)

    def ob(i, carry):
        ones_b[i] = one
        return carry

    lax.fori_loop(0, CH, ob, 0)
    _zero_acc_slice(zbuf, acc, sid)
    plsc.subcore_barrier()

    def fire_s(j, carry):
        pltpu.async_copy(ones_b, acc.at[dstv.at[j]], sem_s, add=True)
        return carry

    lax.fori_loop(0, NCH, fire_s, 0)

    def drain_s(j, carry):
        pltpu.make_async_copy(ones_b, acc.at[dstv.at[j]], sem_s).wait()
        return carry

    lax.fori_loop(0, NCH, drain_s, 0)
    plsc.subcore_barrier()
    base = sid * RPT
    pltpu.sync_copy(acc.at[pl.ds(base, RPT)],
                    out_hbm.at[cid, pl.ds(base, RPT)])


@functools.partial(
    pl.kernel,
    out_type=jax.ShapeDtypeStruct((NCORE, NP, F), jnp.float32),
    mesh=_mesh,
    compiler_params=pltpu.CompilerParams(use_tc_tiling_on_sc=False),
    scratch_types=[
        pltpu.VMEM((NCH, CH), jnp.int32),
        pltpu.VMEM((NCH, CH), jnp.int32),
        pltpu.VMEM((EPT, F), jnp.float32),
        pltpu.VMEM((CH, F), jnp.float32),
        pltpu.VMEM_SHARED((NP, F), jnp.float32),
        pltpu.VMEM_SHARED((NP, F), jnp.float32),
        pltpu.SemaphoreType.DMA,
        pltpu.SemaphoreType.DMA,
        pltpu.SemaphoreType.DMA,
    ],
)
def _sc_propagate_fused(s_hbm, d2_hbm, p_hbm, src_hbm, dst_hbm, out_hbm,
                        srcv, dstv, rows, zbuf, acc, gtab,
                        sem_g, sem_g2, sem_s):
    cid, sid, tid = _tile_ids()
    pltpu.sync_copy(src_hbm.at[pl.ds(tid * NCH, NCH)], srcv)
    pltpu.sync_copy(dst_hbm.at[pl.ds(tid * NCH, NCH)], dstv)
    base = sid * RPT
    pltpu.async_copy(s_hbm.at[pl.ds(base, RPT)], rows.at[pl.ds(0, RPT)], sem_g)
    pltpu.async_copy(d2_hbm.at[pl.ds(base, RPT)], rows.at[pl.ds(RPT, RPT)],
                     sem_g)
    pltpu.async_copy(p_hbm.at[0, pl.ds(base, RPT)],
                     rows.at[pl.ds(2 * RPT, RPT)], sem_g)
    pltpu.async_copy(p_hbm.at[1, pl.ds(base, RPT)],
                     rows.at[pl.ds(3 * RPT, RPT)], sem_g)
    _zero_acc_slice(zbuf, acc, sid)
    pltpu.make_async_copy(s_hbm.at[pl.ds(0, 4 * RPT)],
                          rows.at[pl.ds(0, 4 * RPT)], sem_g).wait()

    def comb(i, carry):
        for u in range(4):
            r = 4 * i + u
            rows[4 * RPT + r] = rows[r] + rows[RPT + r] * (
                rows[2 * RPT + r] + rows[3 * RPT + r])
        return carry

    lax.fori_loop(0, RPT // 4, comb, 0)
    pltpu.sync_copy(rows.at[pl.ds(4 * RPT, RPT)], gtab.at[pl.ds(base, RPT)])
    plsc.subcore_barrier()

    def fire_gathers(k, sem):
        def fg(j, carry):
            pltpu.async_copy(gtab.at[srcv.at[j]],
                             rows.at[pl.ds(j * CH, CH)], sem)
            return carry
        lax.fori_loop(k * BCH, (k + 1) * BCH, fg, 0)

    def wait_gathers(sem):
        pltpu.make_async_copy(s_hbm.at[pl.ds(0, BCH * CH)],
                              rows.at[pl.ds(0, BCH * CH)], sem).wait()

    sems = (sem_g, sem_g2)
    fire_gathers(0, sems[0])
    fire_gathers(1, sems[1])
    for k in range(NBATCH):
        wait_gathers(sems[k % 2])
        if k + 2 < NBATCH:
            fire_gathers(k + 2, sems[k % 2])

        def fs(j, carry):
            pltpu.async_copy(rows.at[pl.ds(j * CH, CH)], acc.at[dstv.at[j]],
                             sem_s, add=True)
            return carry
        lax.fori_loop(k * BCH, (k + 1) * BCH, fs, 0)
    pltpu.make_async_copy(rows, acc.at[pl.ds(0, EPT)], sem_s).wait()
    plsc.subcore_barrier()
    pltpu.sync_copy(acc.at[pl.ds(base, RPT)],
                    out_hbm.at[cid, pl.ds(base, RPT)])


def _leaky(v):
    return jnp.where(v >= 0, v, 0.02 * v)


BM = 1024


def _prep_body(degp_ref, x_ref, w_ref,
               y0_ref, s1_ref, s2_ref, g3_ref, d1_ref, d2_ref):
    deg = degp_ref[0] + degp_ref[1]
    dinv = jnp.where(deg > 0.0, lax.rsqrt(jnp.maximum(deg, 1e-12)), 0.0)
    y = jnp.dot(x_ref[...], w_ref[...], preferred_element_type=jnp.float32)
    y0_ref[...] = y[:, 0:16]
    s1_ref[...] = y[:, 16:32] * dinv
    s2_ref[...] = y[:, 32:48] * dinv
    g3_ref[...] = y[:, 48:64] * dinv
    d1_ref[...] = dinv
    d2_ref[...] = dinv * dinv


_prep = pl.pallas_call(
    _prep_body,
    grid=(NP // BM,),
    in_specs=[
        pl.BlockSpec((NCORE, BM, F), lambda i: (0, i, 0)),
        pl.BlockSpec((BM, D), lambda i: (i, 0)),
        pl.BlockSpec((D, 4 * F), lambda i: (0, 0)),
    ],
    out_specs=[pl.BlockSpec((BM, F), lambda i: (i, 0))] * 6,
    out_shape=[jax.ShapeDtypeStruct((NP, F), jnp.float32)] * 6,
)


def _finish_proj_body(y0_ref, d1_ref, p_ref, b_ref, w_ref,
                      o0_ref, o1_ref, o2_ref, o3_ref):
    h = y0_ref[...] + d1_ref[...] * (p_ref[0] + p_ref[1]) + b_ref[...]
    h = _leaky(h)
    y = jnp.dot(h, w_ref[...], preferred_element_type=jnp.float32)
    dinv = d1_ref[...]
    o0_ref[...] = y[:, 0:16]
    o1_ref[...] = y[:, 16:32] * dinv
    o2_ref[...] = y[:, 32:48] * dinv
    o3_ref[...] = y[:, 48:64] * dinv


_finish_proj = pl.pallas_call(
    _finish_proj_body,
    grid=(NP // (2 * BM),),
    in_specs=[
        pl.BlockSpec((2 * BM, F), lambda i: (i, 0)),
        pl.BlockSpec((2 * BM, F), lambda i: (i, 0)),
        pl.BlockSpec((NCORE, 2 * BM, F), lambda i: (0, i, 0)),
        pl.BlockSpec((1, F), lambda i: (0, 0)),
        pl.BlockSpec((F, 4 * F), lambda i: (0, 0)),
    ],
    out_specs=[pl.BlockSpec((2 * BM, F), lambda i: (i, 0))] * 4,
    out_shape=[jax.ShapeDtypeStruct((NP, F), jnp.float32)] * 4,
)


def _final_body(y0_ref, d1_ref, p_ref, b3_ref,
                w0_ref, b0_ref, w1_ref, b1_ref, w2_ref, b2_ref, o_ref):
    h = y0_ref[...] + d1_ref[...] * (p_ref[0] + p_ref[1]) + b3_ref[...]
    t = _leaky(jnp.dot(h, w0_ref[...], preferred_element_type=jnp.float32)
               + b0_ref[...])
    t = _leaky(jnp.dot(t, w1_ref[...], preferred_element_type=jnp.float32)
               + b1_ref[...])
    o = (jnp.dot(t, w2_ref[...], preferred_element_type=jnp.float32)
         + b2_ref[...])
    m = jnp.max(o, axis=1, keepdims=True)
    e = o - m
    lse = jnp.log(jnp.sum(jnp.exp(e), axis=1, keepdims=True))
    o_ref[...] = e - lse


_final = pl.pallas_call(
    _final_body,
    grid=(NP // (2 * BM),),
    in_specs=[
        pl.BlockSpec((2 * BM, F), lambda i: (i, 0)),
        pl.BlockSpec((2 * BM, F), lambda i: (i, 0)),
        pl.BlockSpec((NCORE, 2 * BM, F), lambda i: (0, i, 0)),
        pl.BlockSpec((1, F), lambda i: (0, 0)),
        pl.BlockSpec((F, 4 * F), lambda i: (0, 0)),
        pl.BlockSpec((1, 4 * F), lambda i: (0, 0)),
        pl.BlockSpec((4 * F, F), lambda i: (0, 0)),
        pl.BlockSpec((1, F), lambda i: (0, 0)),
        pl.BlockSpec((F, C), lambda i: (0, 0)),
        pl.BlockSpec((1, C), lambda i: (0, 0)),
    ],
    out_specs=pl.BlockSpec((2 * BM, C), lambda i: (i, 0)),
    out_shape=jax.ShapeDtypeStruct((NP, C), jnp.float32),
)


def kernel(x, edge_index, conv1_W, conv1_b, conv2_W, conv2_b,
           conv3_W, conv3_b, mlp_W0, mlp_b0, mlp_W1, mlp_b1, mlp_W2, mlp_b2):
    src = edge_index[0].astype(jnp.int32)
    dst = edge_index[1].astype(jnp.int32)
    epad = jnp.full((EP - E,), PADROW, jnp.int32)
    src2 = jnp.concatenate([src, epad]).reshape(NROW, CH)
    dst2 = jnp.concatenate([dst, epad]).reshape(NROW, CH)
    x_pad = jnp.pad(x, ((0, NP - N), (0, 0)))
    w1 = jnp.concatenate([conv1_W[k] for k in range(4)], axis=1)
    w2 = jnp.concatenate([conv2_W[k] for k in range(4)], axis=1)
    w3 = jnp.concatenate([conv3_W[k] for k in range(4)], axis=1)

    degp = _sc_degree(dst2)
    y0, s1, s2, g3, d1, d2 = _prep(degp, x_pad, w1)
    for wc, bc in ((w2, conv1_b), (w3, conv2_b)):
        p = _sc_propagate(g3, src2, dst2)
        p = _sc_propagate_fused(s2, d2, p, src2, dst2)
        p = _sc_propagate_fused(s1, d2, p, src2, dst2)
        y0, s1, s2, g3 = _finish_proj(y0, d1, p, bc.reshape(1, F), wc)
    p = _sc_propagate(g3, src2, dst2)
    p = _sc_propagate_fused(s2, d2, p, src2, dst2)
    p = _sc_propagate_fused(s1, d2, p, src2, dst2)
    out = _final(y0, d1, p, conv3_b.reshape(1, F),
                 mlp_W0, mlp_b0.reshape(1, 4 * F),
                 mlp_W1, mlp_b1.reshape(1, F),
                 mlp_W2, mlp_b2.reshape(1, C))
    return out[:N]


# speedup vs baseline: 1.0508x; 1.0187x over previous
import functools

import jax
import jax.numpy as jnp
from jax import lax
from jax.experimental import pallas as pl
from jax.experimental.pallas import tpu as pltpu
from jax.experimental.pallas import tpu_sc as plsc

N = 10000
E = 160000
D = 256
C = 64
F = 16
NP = 10240
EP = 163840
CH = 128
NROW = EP // CH
NCORE = 2
NSUB = 16
NT = NCORE * NSUB
EPT = EP // NT
NCH = EPT // CH
RPT = NP // NSUB
NBATCH = 40
BCH = NCH // NBATCH
PADROW = N + 100

_mesh = plsc.VectorSubcoreMesh(
    core_axis_name="c", subcore_axis_name="s",
    num_cores=NCORE, num_subcores=NSUB)


def _tile_ids():
    cid = lax.axis_index("c")
    sid = lax.axis_index("s")
    return cid, sid, cid * NSUB + sid


def _zero_acc_slice(zbuf, acc, sid):
    z = jnp.zeros((F,), jnp.float32)

    def zb(i, carry):
        zbuf[i] = z
        return carry

    lax.fori_loop(0, CH, zb, 0)
    base = sid * RPT
    for r in range(RPT // CH):
        pltpu.sync_copy(zbuf, acc.at[pl.ds(base + r * CH, CH)])


@functools.partial(
    pl.kernel,
    out_type=jax.ShapeDtypeStruct((NCORE, NP, F), jnp.float32),
    mesh=_mesh,
    compiler_params=pltpu.CompilerParams(use_tc_tiling_on_sc=False),
    scratch_types=[
        pltpu.VMEM((NCH, CH), jnp.int32),
        pltpu.VMEM((NCH, CH), jnp.int32),
        pltpu.VMEM((EPT, F), jnp.float32),
        pltpu.VMEM((CH, F), jnp.float32),
        pltpu.VMEM_SHARED((NP, F), jnp.float32),
        pltpu.VMEM_SHARED((NP, F), jnp.float32),
        pltpu.SemaphoreType.DMA,
        pltpu.SemaphoreType.DMA,
        pltpu.SemaphoreType.DMA,
    ],
)
def _sc_propagate(g_hbm, src_hbm, dst_hbm, out_hbm,
                  srcv, dstv, rows, zbuf, acc, gtab, sem_g, sem_g2, sem_s):
    cid, sid, tid = _tile_ids()
    pltpu.sync_copy(src_hbm.at[pl.ds(tid * NCH, NCH)], srcv)
    pltpu.sync_copy(dst_hbm.at[pl.ds(tid * NCH, NCH)], dstv)
    base = sid * RPT
    pltpu.async_copy(g_hbm.at[pl.ds(base, RPT)], gtab.at[pl.ds(base, RPT)],
                     sem_g)
    _zero_acc_slice(zbuf, acc, sid)
    pltpu.make_async_copy(g_hbm.at[pl.ds(base, RPT)],
                          gtab.at[pl.ds(base, RPT)], sem_g).wait()
    plsc.subcore_barrier()

    def fire_gathers(k, sem):
        def fg(j, carry):
            pltpu.async_copy(gtab.at[srcv.at[j]],
                             rows.at[pl.ds(j * CH, CH)], sem)
            return carry
        lax.fori_loop(k * BCH, (k + 1) * BCH, fg, 0)

    def wait_gathers(sem):
        pltpu.make_async_copy(g_hbm.at[pl.ds(0, BCH * CH)],
                              rows.at[pl.ds(0, BCH * CH)], sem).wait()

    sems = (sem_g, sem_g2)
    fire_gathers(0, sems[0])
    fire_gathers(1, sems[1])
    for k in range(NBATCH):
        wait_gathers(sems[k % 2])
        if k + 2 < NBATCH:
            fire_gathers(k + 2, sems[k % 2])

        def fs(j, carry):
            pltpu.async_copy(rows.at[pl.ds(j * CH, CH)], acc.at[dstv.at[j]],
                             sem_s, add=True)
            return carry
        lax.fori_loop(k * BCH, (k + 1) * BCH, fs, 0)
    pltpu.make_async_copy(rows, acc.at[pl.ds(0, EPT)], sem_s).wait()
    plsc.subcore_barrier()
    base = sid * RPT
    pltpu.sync_copy(acc.at[pl.ds(base, RPT)],
                    out_hbm.at[cid, pl.ds(base, RPT)])


@functools.partial(
    pl.kernel,
    out_type=jax.ShapeDtypeStruct((NCORE, NP, F), jnp.float32),
    mesh=_mesh,
    compiler_params=pltpu.CompilerParams(use_tc_tiling_on_sc=False),
    scratch_types=[
        pltpu.VMEM((NCH, CH), jnp.int32),
        pltpu.VMEM((CH, F), jnp.float32),
        pltpu.VMEM((CH, F), jnp.float32),
        pltpu.VMEM_SHARED((NP, F), jnp.float32),
        pltpu.SemaphoreType.DMA,
    ],
)
def _sc_degree(dst_hbm, out_hbm, dstv, ones_b, zbuf, acc, sem_s):
    cid, sid, tid = _tile_ids()
    pltpu.sync_copy(dst_hbm.at[pl.ds(tid * NCH, NCH)], dstv)
    one = jnp.ones((F,), jnp.float32)

    def ob(i, carry):
        ones_b[i] = one
        return carry

    lax.fori_loop(0, CH, ob, 0)
    _zero_acc_slice(zbuf, acc, sid)
    plsc.subcore_barrier()

    def fire_s(j, carry):
        pltpu.async_copy(ones_b, acc.at[dstv.at[j]], sem_s, add=True)
        return carry

    lax.fori_loop(0, NCH, fire_s, 0)

    def drain_s(j, carry):
        pltpu.make_async_copy(ones_b, acc.at[dstv.at[j]], sem_s).wait()
        return carry

    lax.fori_loop(0, NCH, drain_s, 0)
    plsc.subcore_barrier()
    base = sid * RPT
    pltpu.sync_copy(acc.at[pl.ds(base, RPT)],
                    out_hbm.at[cid, pl.ds(base, RPT)])


@functools.partial(
    pl.kernel,
    out_type=jax.ShapeDtypeStruct((NCORE, NP, F), jnp.float32),
    mesh=_mesh,
    compiler_params=pltpu.CompilerParams(use_tc_tiling_on_sc=False),
    scratch_types=[
        pltpu.VMEM((NCH, CH), jnp.int32),
        pltpu.VMEM((NCH, CH), jnp.int32),
        pltpu.VMEM((EPT, F), jnp.float32),
        pltpu.VMEM((CH, F), jnp.float32),
        pltpu.VMEM_SHARED((NP, F), jnp.float32),
        pltpu.VMEM_SHARED((NP, F), jnp.float32),
        pltpu.SemaphoreType.DMA,
        pltpu.SemaphoreType.DMA,
        pltpu.SemaphoreType.DMA,
    ],
)
def _sc_propagate_fused(s_hbm, d2_hbm, p_hbm, src_hbm, dst_hbm, out_hbm,
                        srcv, dstv, rows, zbuf, acc, gtab,
                        sem_g, sem_g2, sem_s):
    cid, sid, tid = _tile_ids()
    pltpu.sync_copy(src_hbm.at[pl.ds(tid * NCH, NCH)], srcv)
    pltpu.sync_copy(dst_hbm.at[pl.ds(tid * NCH, NCH)], dstv)
    base = sid * RPT
    pltpu.async_copy(s_hbm.at[pl.ds(base, RPT)], rows.at[pl.ds(0, RPT)], sem_g)
    pltpu.async_copy(d2_hbm.at[pl.ds(base, RPT)], rows.at[pl.ds(RPT, RPT)],
                     sem_g)
    pltpu.async_copy(p_hbm.at[0, pl.ds(base, RPT)],
                     rows.at[pl.ds(2 * RPT, RPT)], sem_g)
    pltpu.async_copy(p_hbm.at[1, pl.ds(base, RPT)],
                     rows.at[pl.ds(3 * RPT, RPT)], sem_g)
    _zero_acc_slice(zbuf, acc, sid)
    pltpu.make_async_copy(s_hbm.at[pl.ds(0, 4 * RPT)],
                          rows.at[pl.ds(0, 4 * RPT)], sem_g).wait()

    def comb(i, carry):
        for u in range(4):
            r = 4 * i + u
            rows[4 * RPT + r] = rows[r] + rows[RPT + r] * (
                rows[2 * RPT + r] + rows[3 * RPT + r])
        return carry

    lax.fori_loop(0, RPT // 4, comb, 0)
    pltpu.sync_copy(rows.at[pl.ds(4 * RPT, RPT)], gtab.at[pl.ds(base, RPT)])
    plsc.subcore_barrier()

    def fire_gathers(k, sem):
        def fg(j, carry):
            pltpu.async_copy(gtab.at[srcv.at[j]],
                             rows.at[pl.ds(j * CH, CH)], sem)
            return carry
        lax.fori_loop(k * BCH, (k + 1) * BCH, fg, 0)

    def wait_gathers(sem):
        pltpu.make_async_copy(s_hbm.at[pl.ds(0, BCH * CH)],
                              rows.at[pl.ds(0, BCH * CH)], sem).wait()

    sems = (sem_g, sem_g2)
    fire_gathers(0, sems[0])
    fire_gathers(1, sems[1])
    for k in range(NBATCH):
        wait_gathers(sems[k % 2])
        if k + 2 < NBATCH:
            fire_gathers(k + 2, sems[k % 2])

        def fs(j, carry):
            pltpu.async_copy(rows.at[pl.ds(j * CH, CH)], acc.at[dstv.at[j]],
                             sem_s, add=True)
            return carry
        lax.fori_loop(k * BCH, (k + 1) * BCH, fs, 0)
    pltpu.make_async_copy(rows, acc.at[pl.ds(0, EPT)], sem_s).wait()
    plsc.subcore_barrier()
    pltpu.sync_copy(acc.at[pl.ds(base, RPT)],
                    out_hbm.at[cid, pl.ds(base, RPT)])


def _leaky(v):
    return jnp.where(v >= 0, v, 0.02 * v)


BM = 1024


def _prep_body(degp_ref, x_ref, w_ref,
               y0_ref, s1_ref, s2_ref, g3_ref, d1_ref, d2_ref):
    deg = degp_ref[0] + degp_ref[1]
    dinv = jnp.where(deg > 0.0, lax.rsqrt(jnp.maximum(deg, 1e-12)), 0.0)
    y = jnp.dot(x_ref[...], w_ref[...], preferred_element_type=jnp.float32)
    y0_ref[...] = y[:, 0:16]
    s1_ref[...] = y[:, 16:32] * dinv
    s2_ref[...] = y[:, 32:48] * dinv
    g3_ref[...] = y[:, 48:64] * dinv
    d1_ref[...] = dinv
    d2_ref[...] = dinv * dinv


_prep = pl.pallas_call(
    _prep_body,
    grid=(NP // BM,),
    in_specs=[
        pl.BlockSpec((NCORE, BM, F), lambda i: (0, i, 0)),
        pl.BlockSpec((BM, D), lambda i: (i, 0)),
        pl.BlockSpec((D, 4 * F), lambda i: (0, 0)),
    ],
    out_specs=[pl.BlockSpec((BM, F), lambda i: (i, 0))] * 6,
    out_shape=[jax.ShapeDtypeStruct((NP, F), jnp.float32)] * 6,
)


def _finish_proj_body(y0_ref, d1_ref, p_ref, b_ref, w_ref,
                      o0_ref, o1_ref, o2_ref, o3_ref):
    h = y0_ref[...] + d1_ref[...] * (p_ref[0] + p_ref[1]) + b_ref[...]
    h = _leaky(h)
    y = jnp.dot(h, w_ref[...], preferred_element_type=jnp.float32)
    dinv = d1_ref[...]
    o0_ref[...] = y[:, 0:16]
    o1_ref[...] = y[:, 16:32] * dinv
    o2_ref[...] = y[:, 32:48] * dinv
    o3_ref[...] = y[:, 48:64] * dinv


_finish_proj = pl.pallas_call(
    _finish_proj_body,
    grid=(NP // (2 * BM),),
    in_specs=[
        pl.BlockSpec((2 * BM, F), lambda i: (i, 0)),
        pl.BlockSpec((2 * BM, F), lambda i: (i, 0)),
        pl.BlockSpec((NCORE, 2 * BM, F), lambda i: (0, i, 0)),
        pl.BlockSpec((1, F), lambda i: (0, 0)),
        pl.BlockSpec((F, 4 * F), lambda i: (0, 0)),
    ],
    out_specs=[pl.BlockSpec((2 * BM, F), lambda i: (i, 0))] * 4,
    out_shape=[jax.ShapeDtypeStruct((NP, F), jnp.float32)] * 4,
)


def _final_body(y0_ref, d1_ref, p_ref, b3_ref,
                w0_ref, b0_ref, w1_ref, b1_ref, w2_ref, b2_ref, o_ref):
    h = y0_ref[...] + d1_ref[...] * (p_ref[0] + p_ref[1]) + b3_ref[...]
    t = _leaky(jnp.dot(h, w0_ref[...], preferred_element_type=jnp.float32)
               + b0_ref[...])
    t = _leaky(jnp.dot(t, w1_ref[...], preferred_element_type=jnp.float32)
               + b1_ref[...])
    o = (jnp.dot(t, w2_ref[...], preferred_element_type=jnp.float32)
         + b2_ref[...])
    m = jnp.max(o, axis=1, keepdims=True)
    e = o - m
    lse = jnp.log(jnp.sum(jnp.exp(e), axis=1, keepdims=True))
    o_ref[...] = e - lse


_final = pl.pallas_call(
    _final_body,
    grid=(NP // (2 * BM),),
    in_specs=[
        pl.BlockSpec((2 * BM, F), lambda i: (i, 0)),
        pl.BlockSpec((2 * BM, F), lambda i: (i, 0)),
        pl.BlockSpec((NCORE, 2 * BM, F), lambda i: (0, i, 0)),
        pl.BlockSpec((1, F), lambda i: (0, 0)),
        pl.BlockSpec((F, 4 * F), lambda i: (0, 0)),
        pl.BlockSpec((1, 4 * F), lambda i: (0, 0)),
        pl.BlockSpec((4 * F, F), lambda i: (0, 0)),
        pl.BlockSpec((1, F), lambda i: (0, 0)),
        pl.BlockSpec((F, C), lambda i: (0, 0)),
        pl.BlockSpec((1, C), lambda i: (0, 0)),
    ],
    out_specs=pl.BlockSpec((2 * BM, C), lambda i: (i, 0)),
    out_shape=jax.ShapeDtypeStruct((NP, C), jnp.float32),
)


def kernel(x, edge_index, conv1_W, conv1_b, conv2_W, conv2_b,
           conv3_W, conv3_b, mlp_W0, mlp_b0, mlp_W1, mlp_b1, mlp_W2, mlp_b2):
    src = edge_index[0].astype(jnp.int32)
    dst = edge_index[1].astype(jnp.int32)
    epad = jnp.full((EP - E,), PADROW, jnp.int32)
    src2 = jnp.concatenate([src, epad]).reshape(NROW, CH)
    dst2 = jnp.concatenate([dst, epad]).reshape(NROW, CH)
    x_pad = jnp.pad(x, ((0, NP - N), (0, 0)))
    w1 = jnp.concatenate([conv1_W[k] for k in range(4)], axis=1)
    w2 = jnp.concatenate([conv2_W[k] for k in range(4)], axis=1)
    w3 = jnp.concatenate([conv3_W[k] for k in range(4)], axis=1)

    degp = _sc_degree(dst2)
    y0, s1, s2, g3, d1, d2 = _prep(degp, x_pad, w1)
    for wc, bc in ((w2, conv1_b), (w3, conv2_b)):
        p = _sc_propagate(g3, src2, dst2)
        p = _sc_propagate_fused(s2, d2, p, src2, dst2)
        p = _sc_propagate_fused(s1, d2, p, src2, dst2)
        y0, s1, s2, g3 = _finish_proj(y0, d1, p, bc.reshape(1, F), wc)
    p = _sc_propagate(g3, src2, dst2)
    p = _sc_propagate_fused(s2, d2, p, src2, dst2)
    p = _sc_propagate_fused(s1, d2, p, src2, dst2)
    out = _final(y0, d1, p, conv3_b.reshape(1, F),
                 mlp_W0, mlp_b0.reshape(1, 4 * F),
                 mlp_W1, mlp_b1.reshape(1, F),
                 mlp_W2, mlp_b2.reshape(1, C))
    return out[:N]
